# Initial kernel scaffold; baseline (speedup 1.0000x reference)
import jax
import jax.numpy as jnp
from jax.experimental import pallas as pl


def kernel(x, y, token_table, pos_table, parent_table):
    raise NotImplementedError("write your pallas kernel here")


# speedup vs baseline: 5.0712x; 5.0712x over previous
import functools

import jax
import jax.numpy as jnp
from jax import lax
from jax.experimental import pallas as pl
from jax.experimental.pallas import tpu as pltpu
from jax.experimental.pallas import tpu_sc as plsc

MAXLEN = 200
EMBED_DIM = 128
BATCH = 1024
NC = 2
NS = 16
NW = NC * NS
SEQ_PER_W = BATCH // NW
HALF = MAXLEN // 2
LANES = 16


def _sc_embed(x3, y3, token_table, pos_table, parent_table):
    mesh = plsc.VectorSubcoreMesh(core_axis_name="c", subcore_axis_name="s")

    @functools.partial(
        pl.kernel,
        out_type=jax.ShapeDtypeStruct((BATCH, MAXLEN, EMBED_DIM), jnp.float32),
        mesh=mesh,
        scratch_types=[
            pltpu.VMEM((2, HALF), jnp.int32),
            pltpu.VMEM((2, HALF), jnp.int32),
            pltpu.VMEM((MAXLEN, EMBED_DIM), jnp.float32),
            pltpu.VMEM((MAXLEN, EMBED_DIM), jnp.float32),
            pltpu.VMEM((MAXLEN, EMBED_DIM), jnp.float32),
            pltpu.SemaphoreType.DMA,
        ],
    )
    def k(x_hbm, y_hbm, tok_hbm, pos_hbm, par_hbm, out_hbm,
          idx_x, idx_y, tok_v, par_v, pos_v, sem):
        wid = lax.axis_index("s") * NC + lax.axis_index("c")
        pltpu.sync_copy(pos_hbm, pos_v)

        @pl.loop(0, SEQ_PER_W)
        def _seq(s):
            b = wid * SEQ_PER_W + s
            pltpu.sync_copy(x_hbm.at[b], idx_x)
            pltpu.sync_copy(y_hbm.at[b], idx_y)
            cps = []
            for j in range(2):
                cps.append(pltpu.async_copy(
                    tok_hbm.at[idx_x.at[j]],
                    tok_v.at[pl.ds(j * HALF, HALF)], sem))
                cps.append(pltpu.async_copy(
                    par_hbm.at[idx_y.at[j]],
                    par_v.at[pl.ds(j * HALF, HALF)], sem))
            for cp in cps:
                cp.wait()

            @pl.loop(0, MAXLEN)
            def _row(r):
                for cb in range(EMBED_DIM // LANES):
                    sl = pl.ds(cb * LANES, LANES)
                    tok_v[r, sl] = tok_v[r, sl] + par_v[r, sl] + pos_v[r, sl]

            pltpu.sync_copy(tok_v, out_hbm.at[b])

    return k(x3, y3, token_table, pos_table, parent_table)


def kernel(x, y, token_table, pos_table, parent_table):
    x3 = x.reshape(BATCH, 2, HALF)
    y3 = y.reshape(BATCH, 2, HALF)
    return _sc_embed(x3, y3, token_table, pos_table, parent_table)


# speedup vs baseline: 5.3929x; 1.0634x over previous
import functools

import jax
import jax.numpy as jnp
from jax import lax
from jax.experimental import pallas as pl
from jax.experimental.pallas import tpu as pltpu
from jax.experimental.pallas import tpu_sc as plsc

MAXLEN = 200
EMBED_DIM = 128
BATCH = 1024
NC = 2
NS = 16
NW = NC * NS
CHUNK = 50
NCHUNK = BATCH * MAXLEN // CHUNK
CPW = NCHUNK // NW
NBUF = 4
NIT = CPW // NBUF
LANES = 16


def _sc_embed(x2, y2, token_table, pos_table, parent_table):
    mesh = plsc.VectorSubcoreMesh(core_axis_name="c", subcore_axis_name="s")

    @functools.partial(
        pl.kernel,
        out_type=jax.ShapeDtypeStruct((NCHUNK, CHUNK, EMBED_DIM), jnp.float32),
        mesh=mesh,
        scratch_types=[
            pltpu.VMEM((CPW, CHUNK), jnp.int32),
            pltpu.VMEM((CPW, CHUNK), jnp.int32),
            [pltpu.VMEM((CHUNK, EMBED_DIM), jnp.float32) for _ in range(NBUF)],
            [pltpu.VMEM((CHUNK, EMBED_DIM), jnp.float32) for _ in range(NBUF)],
            pltpu.VMEM((MAXLEN, EMBED_DIM), jnp.float32),
            [pltpu.SemaphoreType.DMA for _ in range(NBUF)],
            [pltpu.SemaphoreType.DMA for _ in range(NBUF)],
        ],
    )
    def k(x_hbm, y_hbm, tok_hbm, pos_hbm, par_hbm, out_hbm,
          idx_x, idx_y, tok_v, par_v, pos_v, sem_g, sem_o):
        wid = lax.axis_index("s") * NC + lax.axis_index("c")
        base = wid * CPW
        pltpu.sync_copy(x_hbm.at[pl.ds(base, CPW)], idx_x)
        pltpu.sync_copy(y_hbm.at[pl.ds(base, CPW)], idx_y)
        pltpu.sync_copy(pos_hbm, pos_v)

        def issue(c, q):
            pltpu.async_copy(tok_hbm.at[idx_x.at[c]], tok_v[q], sem_g[q])
            pltpu.async_copy(par_hbm.at[idx_y.at[c]], par_v[q], sem_g[q])

        def wait_gather(q):
            pltpu.make_async_copy(tok_hbm.at[idx_x.at[0]], tok_v[q], sem_g[q]).wait()
            pltpu.make_async_copy(par_hbm.at[idx_y.at[0]], par_v[q], sem_g[q]).wait()

        def wait_out(q):
            pltpu.make_async_copy(tok_v[q], out_hbm.at[0], sem_o[q]).wait()

        issue(0, 0)
        issue(1, 1)

        @pl.loop(0, NIT)
        def _it(it):
            for p in range(NBUF):
                c = it * NBUF + p
                q = (p + 2) % NBUF
                if p < 2:
                    @pl.when(it > 0)
                    def _():
                        wait_out(q)
                    issue(c + 2, q)
                else:
                    @pl.when(it < NIT - 1)
                    def _():
                        wait_out(q)
                        issue(c + 2, q)

                wait_gather(p)

                @pl.loop(0, CHUNK)
                def _row(r):
                    for cb in range(EMBED_DIM // LANES):
                        sl = pl.ds(cb * LANES, LANES)
                        tok_v[p][r, sl] = (tok_v[p][r, sl] + par_v[p][r, sl]
                                           + pos_v[p * CHUNK + r, sl])

                pltpu.async_copy(tok_v[p], out_hbm.at[base + c], sem_o[p])

        for p in range(NBUF):
            wait_out(p)

    return k(x2, y2, token_table, pos_table, parent_table)


def kernel(x, y, token_table, pos_table, parent_table):
    x2 = x.reshape(NCHUNK, CHUNK)
    y2 = y.reshape(NCHUNK, CHUNK)
    out = _sc_embed(x2, y2, token_table, pos_table, parent_table)
    return out.reshape(BATCH, MAXLEN, EMBED_DIM)


# speedup vs baseline: 8.5427x; 1.5841x over previous
import functools

import jax
import jax.numpy as jnp
from jax import lax
from jax.experimental import pallas as pl
from jax.experimental.pallas import tpu as pltpu
from jax.experimental.pallas import tpu_sc as plsc

MAXLEN = 200
EMBED_DIM = 128
BATCH = 1024
NC = 2
NS = 16
NW = NC * NS
CHUNK = 40
SPLITS = MAXLEN // CHUNK
SEQ_PER_W = BATCH // NW
CPW = SEQ_PER_W * SPLITS
NBUF = SPLITS
NIT = CPW // NBUF
LANES = 16


def kernel(x, y, token_table, pos_table, parent_table):
    mesh = plsc.VectorSubcoreMesh(core_axis_name="c", subcore_axis_name="s")

    @functools.partial(
        pl.kernel,
        out_type=jax.ShapeDtypeStruct((BATCH, MAXLEN, EMBED_DIM), jnp.float32),
        mesh=mesh,
        scratch_types=[
            pltpu.VMEM((CPW, CHUNK), jnp.int32),
            pltpu.VMEM((CPW, CHUNK), jnp.int32),
            [pltpu.VMEM((CHUNK, EMBED_DIM), jnp.float32) for _ in range(NBUF)],
            [pltpu.VMEM((CHUNK, EMBED_DIM), jnp.float32) for _ in range(NBUF)],
            pltpu.VMEM((MAXLEN, EMBED_DIM), jnp.float32),
            [pltpu.SemaphoreType.DMA for _ in range(NBUF)],
            [pltpu.SemaphoreType.DMA for _ in range(NBUF)],
        ],
    )
    def k(x_hbm, y_hbm, tok_hbm, pos_hbm, par_hbm, out_hbm,
          idx_x, idx_y, tok_v, par_v, pos_v, sem_g, sem_o):
        wid = lax.axis_index("s") * NC + lax.axis_index("c")
        seq0 = wid * SEQ_PER_W
        chunk0 = wid * CPW
        pltpu.sync_copy(x_hbm.at[pl.ds(chunk0, CPW)], idx_x)
        pltpu.sync_copy(y_hbm.at[pl.ds(chunk0, CPW)], idx_y)
        pltpu.sync_copy(pos_hbm, pos_v)

        def issue(it, p, q):
            c = it * NBUF + p
            pltpu.async_copy(tok_hbm.at[idx_x.at[c]], tok_v[q], sem_g[q])
            pltpu.async_copy(par_hbm.at[idx_y.at[c]], par_v[q], sem_g[q])

        def wait_gather(q):
            pltpu.make_async_copy(
                tok_hbm.at[idx_x.at[0]], tok_v[q], sem_g[q]).wait()
            pltpu.make_async_copy(
                par_hbm.at[idx_y.at[0]], par_v[q], sem_g[q]).wait()

        def wait_out(q):
            pltpu.make_async_copy(
                tok_v[q], out_hbm.at[0, pl.ds(0, CHUNK)], sem_o[q]).wait()

        issue(0, 0, 0)
        issue(0, 1, 1)

        @pl.loop(0, NIT)
        def _it(it):
            for p in range(NBUF):
                q = (p + 2) % NBUF
                it2 = it + (p + 2) // NBUF
                if p < NBUF - 2:
                    @pl.when(it > 0)
                    def _():
                        wait_out(q)
                    issue(it2, q, q)
                else:
                    @pl.when(it < NIT - 1)
                    def _():
                        wait_out(q)
                        issue(it2, q, q)

                wait_gather(p)

                @pl.loop(0, CHUNK, unroll=2)
                def _row(r):
                    for cb in range(EMBED_DIM // LANES):
                        sl = pl.ds(cb * LANES, LANES)
                        plsc.addupdate(
                            tok_v[p].at[r, sl],
                            par_v[p][r, sl] + pos_v[p * CHUNK + r, sl])

                pltpu.async_copy(
                    tok_v[p],
                    out_hbm.at[seq0 + it, pl.ds(p * CHUNK, CHUNK)],
                    sem_o[p])

        for p in range(NBUF):
            wait_out(p)

    x2 = x.reshape(BATCH * SPLITS, CHUNK)
    y2 = y.reshape(BATCH * SPLITS, CHUNK)
    return k(x2, y2, token_table, pos_table, parent_table)


# speedup vs baseline: 8.6590x; 1.0136x over previous
import functools

import jax
import jax.numpy as jnp
from jax import lax
from jax.experimental import pallas as pl
from jax.experimental.pallas import tpu as pltpu
from jax.experimental.pallas import tpu_sc as plsc

MAXLEN = 200
EMBED_DIM = 128
BATCH = 1024
VOCAB_PARENT = 1000
NC = 2
NS = 16
NW = NC * NS
CHUNK = 40
SPLITS = MAXLEN // CHUNK
SEQ_PER_W = BATCH // NW
CPW = SEQ_PER_W * SPLITS
NBUF = SPLITS
NIT = CPW // NBUF
LANES = 16


def kernel(x, y, token_table, pos_table, parent_table):
    mesh = plsc.VectorSubcoreMesh(core_axis_name="c", subcore_axis_name="s")

    @functools.partial(
        pl.kernel,
        out_type=jax.ShapeDtypeStruct((BATCH, MAXLEN, EMBED_DIM), jnp.float32),
        mesh=mesh,
        compiler_params=pltpu.CompilerParams(use_tc_tiling_on_sc=False),
        scratch_types=[
            pltpu.VMEM((CPW, CHUNK), jnp.int32),
            pltpu.VMEM((CPW, CHUNK), jnp.int32),
            [pltpu.VMEM((CHUNK, EMBED_DIM), jnp.float32) for _ in range(NBUF)],
            [pltpu.VMEM((CHUNK, EMBED_DIM), jnp.float32) for _ in range(NBUF)],
            pltpu.VMEM((MAXLEN, EMBED_DIM), jnp.float32),
            [pltpu.SemaphoreType.DMA for _ in range(NBUF)],
            [pltpu.SemaphoreType.DMA for _ in range(NBUF)],
        ],
    )
    def k(x_hbm, y_hbm, tok_hbm, pos_hbm, par_hbm, out_hbm,
          idx_x, idx_y, tok_v, par_v, pos_v, sem_g, sem_o):
        wid = lax.axis_index("s") * NC + lax.axis_index("c")
        seq0 = wid * SEQ_PER_W
        chunk0 = wid * CPW

        pltpu.sync_copy(x_hbm.at[pl.ds(chunk0, CPW)], idx_x)
        pltpu.sync_copy(y_hbm.at[pl.ds(chunk0, CPW)], idx_y)
        pltpu.sync_copy(pos_hbm, pos_v)

        def issue(it, p, q):
            c = it * NBUF + p
            pltpu.async_copy(tok_hbm.at[idx_x.at[c]], tok_v[q], sem_g[q])
            pltpu.async_copy(par_hbm.at[idx_y.at[c]], par_v[q], sem_g[q])

        def wait_gather(q):
            pltpu.make_async_copy(
                tok_hbm.at[idx_x.at[0]], tok_v[q], sem_g[q]).wait()
            pltpu.make_async_copy(
                par_hbm.at[idx_y.at[0]], par_v[q], sem_g[q]).wait()

        def wait_out(q):
            pltpu.make_async_copy(
                tok_v[q], out_hbm.at[0, pl.ds(0, CHUNK)], sem_o[q]).wait()

        issue(0, 0, 0)
        issue(0, 1, 1)

        @pl.loop(0, NIT)
        def _it(it):
            for p in range(NBUF):
                q = (p + 2) % NBUF
                it2 = it + (p + 2) // NBUF
                if p < NBUF - 2:
                    @pl.when(it > 0)
                    def _():
                        wait_out(q)
                    issue(it2, q, q)
                else:
                    @pl.when(it < NIT - 1)
                    def _():
                        wait_out(q)
                        issue(it2, q, q)

                wait_gather(p)

                @pl.loop(0, CHUNK, unroll=2)
                def _row(r):
                    for cb in range(EMBED_DIM // LANES):
                        sl = pl.ds(cb * LANES, LANES)
                        plsc.addupdate(
                            tok_v[p].at[r, sl],
                            par_v[p][r, sl] + pos_v[p * CHUNK + r, sl])

                pltpu.async_copy(
                    tok_v[p],
                    out_hbm.at[seq0 + it, pl.ds(p * CHUNK, CHUNK)],
                    sem_o[p])

        for p in range(NBUF):
            wait_out(p)

    x2 = x.reshape(BATCH * SPLITS, CHUNK)
    y2 = y.reshape(BATCH * SPLITS, CHUNK)
    return k(x2, y2, token_table, pos_table, parent_table)
